# baseline (speedup 1.0000x reference)
import functools

import jax
import jax.numpy as jnp
from jax import lax
from jax.experimental import pallas as pl
from jax.experimental.pallas import tpu as pltpu
from jax.experimental.pallas import tpu_sc as plsc

B, L, D = 2, 2048, 1024
H, DH, P = 16, 64, 8
HP = H * P
TL = 256
CENTER = L / 2 - 0.5
_DHH = DH // 2
_QC = 256
_QG = 16
_NQC = L // _QC


def _fold_body(wo_ref, wi_ref, bo_ref, bi_ref, wv_ref, bv_ref):
    wv_ref[...] = jnp.dot(wo_ref[...], wi_ref[...],
                          preferred_element_type=jnp.float32)
    bv_ref[...] = jnp.dot(bo_ref[...], wi_ref[...],
                          preferred_element_type=jnp.float32) + bi_ref[...]


def _fold(W_vp_o, W_vp_i, b_vp_o, b_vp_i):
    return pl.pallas_call(
        _fold_body,
        out_shape=(jax.ShapeDtypeStruct((D, D), jnp.float32),
                   jax.ShapeDtypeStruct((1, D), jnp.float32)),
    )(W_vp_o, W_vp_i, b_vp_o.reshape(1, D), b_vp_i.reshape(1, D))


def _stage_a_body(x_ref, wv_ref, bv_ref, ws_ref, bs_ref, g_ref,
                  val_ref, i0_ref, c0_ref, c1_ref):
    x = x_ref[0]
    val_ref[0] = (jnp.dot(x, wv_ref[...], preferred_element_type=jnp.float32)
                  + bv_ref[...])
    acts = (jnp.dot(x, ws_ref[...], preferred_element_type=jnp.float32)
            + bs_ref[...])
    so_x = acts[:, :HP]
    so_y = acts[:, HP:2 * HP]
    lg = acts[:, 2 * HP:3 * HP]
    m = jnp.max(lg, axis=-1, keepdims=True)
    e = jnp.exp(lg - m)
    gs = jnp.dot(e, g_ref[...], preferred_element_type=jnp.float32)
    aw = e / gs
    wx = jnp.maximum(0.0, 1.0 - jnp.abs(so_x))
    py = CENTER + so_y
    y0f = jnp.floor(py)
    t = py - y0f
    y0 = y0f.astype(jnp.int32)
    v0 = ((y0 >= 0) & (y0 <= L - 1)).astype(jnp.float32)
    v1 = ((y0 >= -1) & (y0 <= L - 2)).astype(jnp.float32)
    awx = aw * wx
    c0 = awx * (1.0 - t) * v0
    c1 = awx * t * v1
    y0cf = jnp.clip(y0f, 0.0, float(L - 1))
    i0_ref[0] = jnp.transpose(y0cf)
    c0_ref[0] = jnp.transpose(c0)
    c1_ref[0] = jnp.transpose(c1)


def _stage_a(x, Wv, bv, W_s, b_s, G):
    grid = (B, L // TL)
    return pl.pallas_call(
        _stage_a_body,
        grid=grid,
        in_specs=[
            pl.BlockSpec((1, TL, D), lambda b, i: (b, i, 0)),
            pl.BlockSpec((D, D), lambda b, i: (0, 0)),
            pl.BlockSpec((1, D), lambda b, i: (0, 0)),
            pl.BlockSpec((D, 3 * HP), lambda b, i: (0, 0)),
            pl.BlockSpec((1, 3 * HP), lambda b, i: (0, 0)),
            pl.BlockSpec((HP, HP), lambda b, i: (0, 0)),
        ],
        out_specs=(
            pl.BlockSpec((1, TL, D), lambda b, i: (b, i, 0)),
            pl.BlockSpec((1, HP, TL), lambda b, i: (b, 0, i)),
            pl.BlockSpec((1, HP, TL), lambda b, i: (b, 0, i)),
            pl.BlockSpec((1, HP, TL), lambda b, i: (b, 0, i)),
        ),
        out_shape=(
            jax.ShapeDtypeStruct((B, L, D), jnp.float32),
            jax.ShapeDtypeStruct((B, HP, L), jnp.float32),
            jax.ShapeDtypeStruct((B, HP, L), jnp.float32),
            jax.ShapeDtypeStruct((B, HP, L), jnp.float32),
        ),
    )(x, Wv, bv, W_s, b_s, G)


def _sc_body(val_hbm, i0_hbm, c0_hbm, c1_hbm, out_hbm,
             table, ibuf, cb0, cb1, obuf):
    b = lax.axis_index("c")
    h = lax.axis_index("s")

    pltpu.sync_copy(i0_hbm.at[b, pl.ds(h * P * L, P * L)], ibuf)
    pltpu.sync_copy(c0_hbm.at[b, pl.ds(h * P * L, P * L)], cb0)
    pltpu.sync_copy(c1_hbm.at[b, pl.ds(h * P * L, P * L)], cb1)

    for half in range(2):
        off = half * _DHH
        pltpu.sync_copy(val_hbm.at[b, :, h, pl.ds(off, _DHH)], table)

        def qchunk(qc, carry):
            qb = qc * _QC

            def query(ql, carry2):
                qi = qb + ql
                acc0 = jnp.zeros((16,), jnp.float32)
                acc1 = jnp.zeros((16,), jnp.float32)
                for p in range(P):
                    y0s = ibuf[pl.ds(p * L + qi, 1)][0].astype(jnp.int32)
                    y1s = jnp.minimum(y0s + 1, L - 1)
                    b0 = jnp.broadcast_to(cb0[pl.ds(p * L + qi, 1)], (16,))
                    b1 = jnp.broadcast_to(cb1[pl.ds(p * L + qi, 1)], (16,))
                    acc0 = (acc0 + b0 * table[y0s, pl.ds(0, 16)]
                            + b1 * table[y1s, pl.ds(0, 16)])
                    acc1 = (acc1 + b0 * table[y0s, pl.ds(16, 16)]
                            + b1 * table[y1s, pl.ds(16, 16)])
                obuf[ql, pl.ds(0, 16)] = acc0
                obuf[ql, pl.ds(16, 16)] = acc1
                return carry2

            lax.fori_loop(0, _QC, query, 0)
            pltpu.sync_copy(
                obuf, out_hbm.at[b, pl.ds(qb, _QC), h, pl.ds(off, _DHH)])
            return carry

        lax.fori_loop(0, _NQC, qchunk, 0)


def _stage_b(value, i0, c0, c1):
    mesh = plsc.VectorSubcoreMesh(core_axis_name="c", subcore_axis_name="s")
    fn = pl.kernel(
        _sc_body,
        out_type=jax.ShapeDtypeStruct((B, L, H, DH), jnp.float32),
        mesh=mesh,
        scratch_types=[
            pltpu.VMEM((L, _DHH), jnp.float32),
            pltpu.VMEM((P * L,), jnp.float32),
            pltpu.VMEM((P * L,), jnp.float32),
            pltpu.VMEM((P * L,), jnp.float32),
            pltpu.VMEM((_QC, _DHH), jnp.float32),
        ],
        compiler_params=pltpu.CompilerParams(use_tc_tiling_on_sc=False),
    )
    return fn(value.reshape(B, L, H, DH),
              i0.reshape(B, HP * L), c0.reshape(B, HP * L),
              c1.reshape(B, HP * L))


def _stage_c_body(s_ref, x_ref, wi_ref, bi_ref, wo_ref, bo_ref, out_ref):
    y = (jnp.dot(s_ref[0], wi_ref[...], preferred_element_type=jnp.float32)
         + bi_ref[...] + x_ref[0])
    out_ref[0] = (jnp.dot(y, wo_ref[...], preferred_element_type=jnp.float32)
                  + bo_ref[...])


def _stage_c(sampled, x, W_op_i, b_op_i, W_op_o, b_op_o):
    grid = (B, L // TL)
    return pl.pallas_call(
        _stage_c_body,
        grid=grid,
        in_specs=[
            pl.BlockSpec((1, TL, D), lambda b, i: (b, i, 0)),
            pl.BlockSpec((1, TL, D), lambda b, i: (b, i, 0)),
            pl.BlockSpec((D, D), lambda b, i: (0, 0)),
            pl.BlockSpec((1, D), lambda b, i: (0, 0)),
            pl.BlockSpec((D, D), lambda b, i: (0, 0)),
            pl.BlockSpec((1, D), lambda b, i: (0, 0)),
        ],
        out_specs=pl.BlockSpec((1, TL, D), lambda b, i: (b, i, 0)),
        out_shape=jax.ShapeDtypeStruct((B, L, D), jnp.float32),
    )(sampled, x, W_op_i, b_op_i.reshape(1, D), W_op_o, b_op_o.reshape(1, D))


def kernel(x, W_vp_o, b_vp_o, W_so, b_so, W_aw, b_aw, W_vp_i, b_vp_i,
           W_op_i, b_op_i, W_op_o, b_op_o):
    Wv, bv = _fold(W_vp_o, W_vp_i, b_vp_o, b_vp_i)
    W_s = jnp.concatenate([W_so[:, 0::2], W_so[:, 1::2], W_aw], axis=1)
    b_s = jnp.concatenate([b_so[0::2][None], b_so[1::2][None], b_aw[None]],
                          axis=1)
    gi = jnp.arange(HP) // P
    G = (gi[:, None] == gi[None, :]).astype(jnp.float32)
    value, i0, c0, c1 = _stage_a(x, Wv, bv, W_s, b_s, G)
    sampled = _stage_b(value, i0, c0, c1)
    return _stage_c(sampled.reshape(B, L, D), x, W_op_i, b_op_i,
                    W_op_o, b_op_o)


# speedup vs baseline: 1.1469x; 1.1469x over previous
import functools

import jax
import jax.numpy as jnp
from jax import lax
from jax.experimental import pallas as pl
from jax.experimental.pallas import tpu as pltpu
from jax.experimental.pallas import tpu_sc as plsc

B, L, D = 2, 2048, 1024
H, DH, P = 16, 64, 8
HP = H * P
TL = 256
CENTER = L / 2 - 0.5
_DHH = DH // 2
_QC = 256
_QG = 16
_NQC = L // _QC


def _fold_body(wo_ref, wi_ref, bo_ref, bi_ref, wv_ref, bv_ref):
    wv_ref[...] = jnp.dot(wo_ref[...], wi_ref[...],
                          preferred_element_type=jnp.float32)
    bv_ref[...] = jnp.dot(bo_ref[...], wi_ref[...],
                          preferred_element_type=jnp.float32) + bi_ref[...]


def _fold(W_vp_o, W_vp_i, b_vp_o, b_vp_i):
    return pl.pallas_call(
        _fold_body,
        out_shape=(jax.ShapeDtypeStruct((D, D), jnp.float32),
                   jax.ShapeDtypeStruct((1, D), jnp.float32)),
    )(W_vp_o, W_vp_i, b_vp_o.reshape(1, D), b_vp_i.reshape(1, D))


def _stage_a_body(x_ref, wv_ref, bv_ref, ws_ref, bs_ref, g_ref,
                  val_ref, i0_ref, c0_ref, c1_ref):
    x = x_ref[0]
    val_ref[0] = (jnp.dot(x, wv_ref[...], preferred_element_type=jnp.float32)
                  + bv_ref[...])
    acts = (jnp.dot(x, ws_ref[...], preferred_element_type=jnp.float32)
            + bs_ref[...])
    so_x = acts[:, :HP]
    so_y = acts[:, HP:2 * HP]
    lg = acts[:, 2 * HP:3 * HP]
    m = jnp.max(lg, axis=-1, keepdims=True)
    e = jnp.exp(lg - m)
    gs = jnp.dot(e, g_ref[...], preferred_element_type=jnp.float32)
    aw = e / gs
    wx = jnp.maximum(0.0, 1.0 - jnp.abs(so_x))
    py = CENTER + so_y
    y0f = jnp.floor(py)
    t = py - y0f
    y0 = y0f.astype(jnp.int32)
    v0 = ((y0 >= 0) & (y0 <= L - 1)).astype(jnp.float32)
    v1 = ((y0 >= -1) & (y0 <= L - 2)).astype(jnp.float32)
    awx = aw * wx
    c0 = awx * (1.0 - t) * v0
    c1 = awx * t * v1
    y0cf = jnp.clip(y0f, 0.0, float(L - 1))
    i0_ref[0] = jnp.transpose(y0cf)
    c0_ref[0] = jnp.transpose(c0)
    c1_ref[0] = jnp.transpose(c1)


def _stage_a(x, Wv, bv, W_s, b_s, G):
    grid = (B, L // TL)
    return pl.pallas_call(
        _stage_a_body,
        grid=grid,
        in_specs=[
            pl.BlockSpec((1, TL, D), lambda b, i: (b, i, 0)),
            pl.BlockSpec((D, D), lambda b, i: (0, 0)),
            pl.BlockSpec((1, D), lambda b, i: (0, 0)),
            pl.BlockSpec((D, 3 * HP), lambda b, i: (0, 0)),
            pl.BlockSpec((1, 3 * HP), lambda b, i: (0, 0)),
            pl.BlockSpec((HP, HP), lambda b, i: (0, 0)),
        ],
        out_specs=(
            pl.BlockSpec((1, TL, D), lambda b, i: (b, i, 0)),
            pl.BlockSpec((1, HP, TL), lambda b, i: (b, 0, i)),
            pl.BlockSpec((1, HP, TL), lambda b, i: (b, 0, i)),
            pl.BlockSpec((1, HP, TL), lambda b, i: (b, 0, i)),
        ),
        out_shape=(
            jax.ShapeDtypeStruct((B, L, D), jnp.float32),
            jax.ShapeDtypeStruct((B, HP, L), jnp.float32),
            jax.ShapeDtypeStruct((B, HP, L), jnp.float32),
            jax.ShapeDtypeStruct((B, HP, L), jnp.float32),
        ),
    )(x, Wv, bv, W_s, b_s, G)


def _sc_body(val_hbm, i0_hbm, c0_hbm, c1_hbm, out_hbm,
             table, ibuf, cb0, cb1, obuf):
    b = lax.axis_index("c")
    h = lax.axis_index("s")

    pltpu.sync_copy(i0_hbm.at[b, pl.ds(h * P * L, P * L)], ibuf)
    pltpu.sync_copy(c0_hbm.at[b, pl.ds(h * P * L, P * L)], cb0)
    pltpu.sync_copy(c1_hbm.at[b, pl.ds(h * P * L, P * L)], cb1)

    for half in range(2):
        off = half * _DHH
        pltpu.sync_copy(val_hbm.at[b, :, h, pl.ds(off, _DHH)],
                        table.at[pl.ds(0, L), :])
        table[L, pl.ds(0, 16)] = jnp.zeros((16,), jnp.float32)
        table[L, pl.ds(16, 16)] = jnp.zeros((16,), jnp.float32)

        def qchunk(qc, carry):
            qb = qc * _QC

            def group(g, carry2):
                qg = qb + g * _QG
                ivv = [ibuf[pl.ds(p * L + qg, _QG)].astype(jnp.int32)
                       for p in range(P)]
                cv0 = [cb0[pl.ds(p * L + qg, _QG)] for p in range(P)]
                cv1 = [cb1[pl.ds(p * L + qg, _QG)] for p in range(P)]
                for q in range(_QG):
                    acc0 = jnp.zeros((16,), jnp.float32)
                    acc1 = jnp.zeros((16,), jnp.float32)
                    for p in range(P):
                        y0s = ivv[p][q]
                        y1s = y0s + 1
                        b0 = jnp.broadcast_to(
                            lax.slice(cv0[p], (q,), (q + 1,)), (16,))
                        b1 = jnp.broadcast_to(
                            lax.slice(cv1[p], (q,), (q + 1,)), (16,))
                        acc0 = (acc0 + b0 * table[y0s, pl.ds(0, 16)]
                                + b1 * table[y1s, pl.ds(0, 16)])
                        acc1 = (acc1 + b0 * table[y0s, pl.ds(16, 16)]
                                + b1 * table[y1s, pl.ds(16, 16)])
                    ql = g * _QG + q
                    obuf[ql, pl.ds(0, 16)] = acc0
                    obuf[ql, pl.ds(16, 16)] = acc1
                return carry2

            lax.fori_loop(0, _QC // _QG, group, 0)
            pltpu.sync_copy(
                obuf, out_hbm.at[b, pl.ds(qb, _QC), h, pl.ds(off, _DHH)])
            return carry

        lax.fori_loop(0, _NQC, qchunk, 0)


def _stage_b(value, i0, c0, c1):
    mesh = plsc.VectorSubcoreMesh(core_axis_name="c", subcore_axis_name="s")
    fn = pl.kernel(
        _sc_body,
        out_type=jax.ShapeDtypeStruct((B, L, H, DH), jnp.float32),
        mesh=mesh,
        scratch_types=[
            pltpu.VMEM((L + 1, _DHH), jnp.float32),
            pltpu.VMEM((P * L,), jnp.float32),
            pltpu.VMEM((P * L,), jnp.float32),
            pltpu.VMEM((P * L,), jnp.float32),
            pltpu.VMEM((_QC, _DHH), jnp.float32),
        ],
        compiler_params=pltpu.CompilerParams(use_tc_tiling_on_sc=False),
    )
    return fn(value.reshape(B, L, H, DH),
              i0.reshape(B, HP * L), c0.reshape(B, HP * L),
              c1.reshape(B, HP * L))


def _stage_c_body(s_ref, x_ref, wi_ref, bi_ref, wo_ref, bo_ref, out_ref):
    y = (jnp.dot(s_ref[0], wi_ref[...], preferred_element_type=jnp.float32)
         + bi_ref[...] + x_ref[0])
    out_ref[0] = (jnp.dot(y, wo_ref[...], preferred_element_type=jnp.float32)
                  + bo_ref[...])


def _stage_c(sampled, x, W_op_i, b_op_i, W_op_o, b_op_o):
    grid = (B, L // TL)
    return pl.pallas_call(
        _stage_c_body,
        grid=grid,
        in_specs=[
            pl.BlockSpec((1, TL, D), lambda b, i: (b, i, 0)),
            pl.BlockSpec((1, TL, D), lambda b, i: (b, i, 0)),
            pl.BlockSpec((D, D), lambda b, i: (0, 0)),
            pl.BlockSpec((1, D), lambda b, i: (0, 0)),
            pl.BlockSpec((D, D), lambda b, i: (0, 0)),
            pl.BlockSpec((1, D), lambda b, i: (0, 0)),
        ],
        out_specs=pl.BlockSpec((1, TL, D), lambda b, i: (b, i, 0)),
        out_shape=jax.ShapeDtypeStruct((B, L, D), jnp.float32),
    )(sampled, x, W_op_i, b_op_i.reshape(1, D), W_op_o, b_op_o.reshape(1, D))


def kernel(x, W_vp_o, b_vp_o, W_so, b_so, W_aw, b_aw, W_vp_i, b_vp_i,
           W_op_i, b_op_i, W_op_o, b_op_o):
    Wv, bv = _fold(W_vp_o, W_vp_i, b_vp_o, b_vp_i)
    W_s = jnp.concatenate([W_so[:, 0::2], W_so[:, 1::2], W_aw], axis=1)
    b_s = jnp.concatenate([b_so[0::2][None], b_so[1::2][None], b_aw[None]],
                          axis=1)
    gi = jnp.arange(HP) // P
    G = (gi[:, None] == gi[None, :]).astype(jnp.float32)
    value, i0, c0, c1 = _stage_a(x, Wv, bv, W_s, b_s, G)
    sampled = _stage_b(value, i0, c0, c1)
    return _stage_c(sampled.reshape(B, L, D), x, W_op_i, b_op_i,
                    W_op_o, b_op_o)
